# Initial kernel scaffold; baseline (speedup 1.0000x reference)
import functools

import jax
import jax.numpy as jnp
from jax import lax
from jax.experimental import pallas as pl
from jax.experimental.pallas import tpu as pltpu
from jax.experimental.pallas import tpu_sc as plsc

N_NODES = 10000
SIGNAL_DIM = 128
KV = 16
N_ROT = 4
P = N_NODES * KV
NW = 32
PW = P // NW
CHUNK = 40
NCHUNK = PW // CHUNK


def _sc_interp(table, i1, i2, i3, w1, w2, w3):
    mesh = plsc.VectorSubcoreMesh(core_axis_name="c", subcore_axis_name="s")

    @functools.partial(
        pl.kernel,
        mesh=mesh,
        out_type=jax.ShapeDtypeStruct((P, SIGNAL_DIM), jnp.float32),
        scratch_types=[
            pltpu.VMEM((PW,), jnp.int32),
            pltpu.VMEM((PW,), jnp.int32),
            pltpu.VMEM((PW,), jnp.int32),
            pltpu.VMEM((PW,), jnp.float32),
            pltpu.VMEM((PW,), jnp.float32),
            pltpu.VMEM((PW,), jnp.float32),
            pltpu.VMEM((CHUNK, SIGNAL_DIM), jnp.float32),
            pltpu.VMEM((CHUNK, SIGNAL_DIM), jnp.float32),
            pltpu.VMEM((CHUNK, SIGNAL_DIM), jnp.float32),
            pltpu.VMEM((CHUNK, SIGNAL_DIM), jnp.float32),
            pltpu.SemaphoreType.DMA,
        ],
    )
    def k(table_hbm, i1_hbm, i2_hbm, i3_hbm, w1_hbm, w2_hbm, w3_hbm, out_hbm,
          i1_v, i2_v, i3_v, w1_v, w2_v, w3_v, r1, r2, r3, x_v, sem):
        wid = lax.axis_index("s") * 2 + lax.axis_index("c")
        base = wid * PW
        pltpu.sync_copy(i1_hbm.at[pl.ds(base, PW)], i1_v)
        pltpu.sync_copy(i2_hbm.at[pl.ds(base, PW)], i2_v)
        pltpu.sync_copy(i3_hbm.at[pl.ds(base, PW)], i3_v)
        pltpu.sync_copy(w1_hbm.at[pl.ds(base, PW)], w1_v)
        pltpu.sync_copy(w2_hbm.at[pl.ds(base, PW)], w2_v)
        pltpu.sync_copy(w3_hbm.at[pl.ds(base, PW)], w3_v)

        def chunk_body(ci, carry):
            c0 = ci * CHUNK
            cp1 = pltpu.async_copy(table_hbm.at[i1_v.at[pl.ds(c0, CHUNK)]], r1, sem)
            cp2 = pltpu.async_copy(table_hbm.at[i2_v.at[pl.ds(c0, CHUNK)]], r2, sem)
            cp3 = pltpu.async_copy(table_hbm.at[i3_v.at[pl.ds(c0, CHUNK)]], r3, sem)
            cp1.wait()
            cp2.wait()
            cp3.wait()

            def pair_body(c, inner):
                a1 = w1_v[c0 + c]
                a2 = w2_v[c0 + c]
                a3 = w3_v[c0 + c]
                for v in range(SIGNAL_DIM // 16):
                    sl = pl.ds(v * 16, 16)
                    x_v[c, sl] = r1[c, sl] * a1 + r2[c, sl] * a2 + r3[c, sl] * a3
                return inner

            lax.fori_loop(0, CHUNK, pair_body, None)
            pltpu.sync_copy(x_v, out_hbm.at[pl.ds(base + c0, CHUNK)])
            return carry

        lax.fori_loop(0, NCHUNK, chunk_body, None)

    return k(table, i1, i2, i3, w1, w2, w3)


def _tc_conv(X, W):
    M = 400
    grid = (N_NODES // M,)

    def body(x_ref, w_ref, o_ref):
        pre = jnp.dot(x_ref[...], w_ref[...], preferred_element_type=jnp.float32)
        pre = jnp.maximum(pre, 0.0)
        best = jnp.zeros((M,), jnp.int32)
        cur = None
        for r in range(N_ROT):
            pr = pre[:, r * 128:(r + 1) * 128]
            nr = jnp.sum(pr * pr, axis=-1)
            if cur is None:
                cur = nr
            else:
                gt = nr > cur
                best = jnp.where(gt, r, best)
                cur = jnp.maximum(nr, cur)
        out = jnp.zeros((M, 128), jnp.float32)
        for r in range(N_ROT):
            pr = pre[:, r * 128:(r + 1) * 128]
            out = out + jnp.where((best == r)[:, None], pr, 0.0)
        o_ref[...] = out

    return pl.pallas_call(
        body,
        grid=grid,
        in_specs=[
            pl.BlockSpec((M, KV * SIGNAL_DIM), lambda i: (i, 0)),
            pl.BlockSpec((KV * SIGNAL_DIM, N_ROT * 128), lambda i: (0, 0)),
        ],
        out_specs=pl.BlockSpec((M, 128), lambda i: (i, 0)),
        out_shape=jax.ShapeDtypeStruct((N_NODES, 128), jnp.float32),
    )(X, W)


def kernel(signal, b_coordinates, kernel):
    s = signal[0]
    b = b_coordinates[0].reshape(-1, 8)
    i1 = b[:, 3].astype(jnp.int32)
    i2 = b[:, 5].astype(jnp.int32)
    i3 = b[:, 7].astype(jnp.int32)
    w1 = b[:, 2]
    w2 = b[:, 4]
    w3 = b[:, 6]

    x = _sc_interp(s, i1, i2, i3, w1, w2, w3)
    X = x.reshape(N_NODES, KV * SIGNAL_DIM)

    krot = jnp.stack([jnp.roll(kernel, -r, axis=1) for r in range(N_ROT)], axis=0)
    krot = krot[:, :, :, 0]
    W = krot.transpose(1, 2, 4, 0, 3).reshape(KV * SIGNAL_DIM, N_ROT * 128)

    out = _tc_conv(X, W)
    return out[None]


# speedup vs baseline: 6.6533x; 6.6533x over previous
import functools

import jax
import jax.numpy as jnp
from jax import lax
from jax.experimental import pallas as pl
from jax.experimental.pallas import tpu as pltpu
from jax.experimental.pallas import tpu_sc as plsc

N_NODES = 10000
SIGNAL_DIM = 128
KV = 16
N_ROT = 4
P = N_NODES * KV
NW = 32
CHUNK = 128
NCHUNK = P // CHUNK
FULL_ROUNDS = NCHUNK // NW
REM = NCHUNK - FULL_ROUNDS * NW


def _sc_interp(table, i1, i2, i3, w1, w2, w3):
    mesh = plsc.VectorSubcoreMesh(core_axis_name="c", subcore_axis_name="s")

    @functools.partial(
        pl.kernel,
        mesh=mesh,
        out_type=jax.ShapeDtypeStruct((P, SIGNAL_DIM), jnp.float32),
        scratch_types=[
            pltpu.VMEM((CHUNK,), jnp.int32),
            pltpu.VMEM((CHUNK,), jnp.int32),
            pltpu.VMEM((CHUNK,), jnp.int32),
            pltpu.VMEM((CHUNK,), jnp.float32),
            pltpu.VMEM((CHUNK,), jnp.float32),
            pltpu.VMEM((CHUNK,), jnp.float32),
            pltpu.VMEM((CHUNK, SIGNAL_DIM), jnp.float32),
            pltpu.VMEM((CHUNK, SIGNAL_DIM), jnp.float32),
            pltpu.VMEM((CHUNK, SIGNAL_DIM), jnp.float32),
            pltpu.VMEM((CHUNK, SIGNAL_DIM), jnp.float32),
            pltpu.SemaphoreType.DMA,
            pltpu.SemaphoreType.DMA,
        ],
    )
    def k(table_hbm, i1_hbm, i2_hbm, i3_hbm, w1_hbm, w2_hbm, w3_hbm, out_hbm,
          i1_v, i2_v, i3_v, w1_v, w2_v, w3_v, r1, r2, r3, x_v, sem_idx, sem_dat):
        wid = lax.axis_index("s") * 2 + lax.axis_index("c")

        def do_chunk(g):
            c0 = g * CHUNK
            ci1 = pltpu.async_copy(i1_hbm.at[pl.ds(c0, CHUNK)], i1_v, sem_idx)
            ci2 = pltpu.async_copy(i2_hbm.at[pl.ds(c0, CHUNK)], i2_v, sem_idx)
            ci3 = pltpu.async_copy(i3_hbm.at[pl.ds(c0, CHUNK)], i3_v, sem_idx)
            cw1 = pltpu.async_copy(w1_hbm.at[pl.ds(c0, CHUNK)], w1_v, sem_dat)
            cw2 = pltpu.async_copy(w2_hbm.at[pl.ds(c0, CHUNK)], w2_v, sem_dat)
            cw3 = pltpu.async_copy(w3_hbm.at[pl.ds(c0, CHUNK)], w3_v, sem_dat)
            ci1.wait()
            ci2.wait()
            ci3.wait()
            cp1 = pltpu.async_copy(table_hbm.at[i1_v], r1, sem_dat)
            cp2 = pltpu.async_copy(table_hbm.at[i2_v], r2, sem_dat)
            cp3 = pltpu.async_copy(table_hbm.at[i3_v], r3, sem_dat)
            cw1.wait()
            cw2.wait()
            cw3.wait()
            cp1.wait()
            cp2.wait()
            cp3.wait()

            def group_body(gg, inner):
                gbase = gg * 16
                av1 = w1_v[pl.ds(gbase, 16)]
                av2 = w2_v[pl.ds(gbase, 16)]
                av3 = w3_v[pl.ds(gbase, 16)]
                for l in range(16):
                    c = gbase + l
                    a1 = av1[l]
                    a2 = av2[l]
                    a3 = av3[l]
                    for v in range(SIGNAL_DIM // 16):
                        sl = pl.ds(v * 16, 16)
                        x_v[c, sl] = r1[c, sl] * a1 + r2[c, sl] * a2 + r3[c, sl] * a3
                return inner

            lax.fori_loop(0, CHUNK // 16, group_body, None)
            pltpu.sync_copy(x_v, out_hbm.at[pl.ds(c0, CHUNK)])

        def round_body(li, carry):
            do_chunk(li * NW + wid)
            return carry

        lax.fori_loop(0, FULL_ROUNDS, round_body, None)

        @pl.when(wid < REM)
        def _():
            do_chunk(FULL_ROUNDS * NW + wid)

    return k(table, i1, i2, i3, w1, w2, w3)


def _tc_conv(X, W):
    M = 400
    grid = (N_NODES // M,)

    def body(x_ref, w_ref, o_ref):
        pre = jnp.dot(x_ref[...].astype(jnp.bfloat16),
                      w_ref[...].astype(jnp.bfloat16),
                      preferred_element_type=jnp.float32)
        pre = jnp.maximum(pre, 0.0)
        best = jnp.zeros((M,), jnp.int32)
        cur = None
        for r in range(N_ROT):
            pr = pre[:, r * 128:(r + 1) * 128]
            nr = jnp.sum(pr * pr, axis=-1)
            if cur is None:
                cur = nr
            else:
                gt = nr > cur
                best = jnp.where(gt, r, best)
                cur = jnp.maximum(nr, cur)
        out = jnp.zeros((M, 128), jnp.float32)
        for r in range(N_ROT):
            pr = pre[:, r * 128:(r + 1) * 128]
            out = out + jnp.where((best == r)[:, None], pr, 0.0)
        o_ref[...] = out

    return pl.pallas_call(
        body,
        grid=grid,
        in_specs=[
            pl.BlockSpec((M, KV * SIGNAL_DIM), lambda i: (i, 0)),
            pl.BlockSpec((KV * SIGNAL_DIM, N_ROT * 128), lambda i: (0, 0)),
        ],
        out_specs=pl.BlockSpec((M, 128), lambda i: (i, 0)),
        out_shape=jax.ShapeDtypeStruct((N_NODES, 128), jnp.float32),
    )(X, W)


def kernel(signal, b_coordinates, kernel):
    s = signal[0]
    b = b_coordinates[0].reshape(-1, 8)
    i1 = b[:, 3].astype(jnp.int32)
    i2 = b[:, 5].astype(jnp.int32)
    i3 = b[:, 7].astype(jnp.int32)
    w1 = b[:, 2]
    w2 = b[:, 4]
    w3 = b[:, 6]

    x = _sc_interp(s, i1, i2, i3, w1, w2, w3)
    X = x.reshape(N_NODES, KV * SIGNAL_DIM)

    krot = jnp.stack([jnp.roll(kernel, -r, axis=1) for r in range(N_ROT)], axis=0)
    krot = krot[:, :, :, 0]
    W = krot.transpose(1, 2, 4, 0, 3).reshape(KV * SIGNAL_DIM, N_ROT * 128)

    out = _tc_conv(X, W)
    return out[None]


# speedup vs baseline: 8.6404x; 1.2987x over previous
import functools

import jax
import jax.numpy as jnp
from jax import lax
from jax.experimental import pallas as pl
from jax.experimental.pallas import tpu as pltpu
from jax.experimental.pallas import tpu_sc as plsc

N_NODES = 10000
SIGNAL_DIM = 128
KV = 16
N_ROT = 4
P = N_NODES * KV
NW = 32
CHUNK = 80
NCHUNK = P // CHUNK
ROUNDS = -(-NCHUNK // NW)
OUTER = (ROUNDS + 2) // 2


def _sc_interp(table, i1, i2, i3, w1, w2, w3):
    mesh = plsc.VectorSubcoreMesh(core_axis_name="c", subcore_axis_name="s")

    nbuf = 2
    vm = pltpu.VMEM
    buf_types = []
    for _ in range(nbuf):
        buf_types += (
            [vm((CHUNK,), jnp.int32)] * 3
            + [vm((CHUNK,), jnp.float32)] * 3
            + [vm((CHUNK, SIGNAL_DIM), jnp.float32)] * 3
            + [vm((CHUNK, SIGNAL_DIM), jnp.float32)]
        )
    sem_types = [pltpu.SemaphoreType.DMA] * (3 * nbuf)

    @functools.partial(
        pl.kernel,
        mesh=mesh,
        out_type=jax.ShapeDtypeStruct((P, SIGNAL_DIM), jnp.float32),
        scratch_types=buf_types + sem_types,
    )
    def k(table_hbm, i1_hbm, i2_hbm, i3_hbm, w1_hbm, w2_hbm, w3_hbm, out_hbm,
          *scr):
        wid = lax.axis_index("s") * 2 + lax.axis_index("c")
        i_hbm = (i1_hbm, i2_hbm, i3_hbm)
        w_hbm = (w1_hbm, w2_hbm, w3_hbm)
        bufs = [scr[b * 10:(b + 1) * 10] for b in range(nbuf)]
        sems = scr[10 * nbuf:]
        sem_idx = sems[0::3]
        sem_dat = sems[1::3]
        sem_out = sems[2::3]

        def load_idx(b, li):
            g = li * NW + wid

            @pl.when(g < NCHUNK)
            def _():
                c0 = g * CHUNK
                for t in range(3):
                    pltpu.async_copy(i_hbm[t].at[pl.ds(c0, CHUNK)],
                                     bufs[b][t], sem_idx[b])
                    pltpu.async_copy(w_hbm[t].at[pl.ds(c0, CHUNK)],
                                     bufs[b][3 + t], sem_dat[b])

        def launch_gather(b, li):
            g = li * NW + wid

            @pl.when(g < NCHUNK)
            def _():
                for t in range(3):
                    pltpu.make_async_copy(i_hbm[t].at[pl.ds(0, CHUNK)],
                                          bufs[b][t], sem_idx[b]).wait()
                for t in range(3):
                    pltpu.async_copy(table_hbm.at[bufs[b][t]],
                                     bufs[b][6 + t], sem_dat[b])

        def compute_store(b, li):
            g = li * NW + wid

            @pl.when(g < NCHUNK)
            def _():
                c0 = g * CHUNK
                x_v = bufs[b][9]
                r1, r2, r3 = bufs[b][6], bufs[b][7], bufs[b][8]
                for t in range(3):
                    pltpu.make_async_copy(w_hbm[t].at[pl.ds(0, CHUNK)],
                                          bufs[b][3 + t], sem_dat[b]).wait()
                for t in range(3):
                    pltpu.make_async_copy(table_hbm.at[bufs[b][t]],
                                          bufs[b][6 + t], sem_dat[b]).wait()

                @pl.when(li >= 2)
                def _():
                    pltpu.make_async_copy(
                        x_v, out_hbm.at[pl.ds(0, CHUNK)], sem_out[b]).wait()

                def group_body(gg, inner):
                    gbase = gg * 16
                    av1 = bufs[b][3][pl.ds(gbase, 16)]
                    av2 = bufs[b][4][pl.ds(gbase, 16)]
                    av3 = bufs[b][5][pl.ds(gbase, 16)]
                    for l in range(16):
                        c = gbase + l
                        a1 = av1[l]
                        a2 = av2[l]
                        a3 = av3[l]
                        for v in range(SIGNAL_DIM // 16):
                            sl = pl.ds(v * 16, 16)
                            x_v[c, sl] = (r1[c, sl] * a1 + r2[c, sl] * a2
                                          + r3[c, sl] * a3)
                    return inner

                lax.fori_loop(0, CHUNK // 16, group_body, None)
                pltpu.async_copy(x_v, out_hbm.at[pl.ds(c0, CHUNK)], sem_out[b])

        load_idx(0, 0)
        launch_gather(0, 0)
        load_idx(1, 1)
        launch_gather(1, 1)

        def outer_body(i, carry):
            li = i * 2
            compute_store(0, li)
            load_idx(0, li + 2)
            launch_gather(0, li + 2)
            compute_store(1, li + 1)
            load_idx(1, li + 3)
            launch_gather(1, li + 3)
            return carry

        lax.fori_loop(0, OUTER, outer_body, None)
        for b in range(nbuf):
            pltpu.make_async_copy(
                bufs[b][9], out_hbm.at[pl.ds(0, CHUNK)], sem_out[b]).wait()

    return k(table, i1, i2, i3, w1, w2, w3)


def _tc_conv(X, W):
    M = 400
    grid = (N_NODES // M,)

    def body(x_ref, w_ref, o_ref):
        pre = jnp.dot(x_ref[...].astype(jnp.bfloat16),
                      w_ref[...].astype(jnp.bfloat16),
                      preferred_element_type=jnp.float32)
        pre = jnp.maximum(pre, 0.0)
        best = jnp.zeros((M,), jnp.int32)
        cur = None
        for r in range(N_ROT):
            pr = pre[:, r * 128:(r + 1) * 128]
            nr = jnp.sum(pr * pr, axis=-1)
            if cur is None:
                cur = nr
            else:
                gt = nr > cur
                best = jnp.where(gt, r, best)
                cur = jnp.maximum(nr, cur)
        out = jnp.zeros((M, 128), jnp.float32)
        for r in range(N_ROT):
            pr = pre[:, r * 128:(r + 1) * 128]
            out = out + jnp.where((best == r)[:, None], pr, 0.0)
        o_ref[...] = out

    return pl.pallas_call(
        body,
        grid=grid,
        in_specs=[
            pl.BlockSpec((M, KV * SIGNAL_DIM), lambda i: (i, 0)),
            pl.BlockSpec((KV * SIGNAL_DIM, N_ROT * 128), lambda i: (0, 0)),
        ],
        out_specs=pl.BlockSpec((M, 128), lambda i: (i, 0)),
        out_shape=jax.ShapeDtypeStruct((N_NODES, 128), jnp.float32),
    )(X, W)


def kernel(signal, b_coordinates, kernel):
    s = signal[0]
    b = b_coordinates[0].reshape(-1, 8)
    i1 = b[:, 3].astype(jnp.int32)
    i2 = b[:, 5].astype(jnp.int32)
    i3 = b[:, 7].astype(jnp.int32)
    w1 = b[:, 2]
    w2 = b[:, 4]
    w3 = b[:, 6]

    x = _sc_interp(s, i1, i2, i3, w1, w2, w3)
    X = x.reshape(N_NODES, KV * SIGNAL_DIM)

    krot = jnp.stack([jnp.roll(kernel, -r, axis=1) for r in range(N_ROT)], axis=0)
    krot = krot[:, :, :, 0]
    W = krot.transpose(1, 2, 4, 0, 3).reshape(KV * SIGNAL_DIM, N_ROT * 128)

    out = _tc_conv(X, W)
    return out[None]


# speedup vs baseline: 9.1951x; 1.0642x over previous
import functools

import jax
import jax.numpy as jnp
from jax import lax
from jax.experimental import pallas as pl
from jax.experimental.pallas import tpu as pltpu
from jax.experimental.pallas import tpu_sc as plsc

N_NODES = 10000
SIGNAL_DIM = 128
KV = 16
N_ROT = 4
P = N_NODES * KV
NW = 32
CHUNK = 80
NCHUNK = P // CHUNK
ROUNDS = -(-NCHUNK // NW)
OUTER = (ROUNDS + 2) // 2


def _sc_interp(table, i1, i2, i3, w1, w2, w3):
    mesh = plsc.VectorSubcoreMesh(core_axis_name="c", subcore_axis_name="s")
    p_n = i1.shape[0]
    nchunk = p_n // CHUNK
    outer = (-(-nchunk // NW) + 1) // 2

    nbuf = 2
    vm = pltpu.VMEM
    buf_types = []
    for _ in range(nbuf):
        buf_types += (
            [vm((CHUNK,), jnp.int32)] * 3
            + [vm((CHUNK,), jnp.float32)] * 3
            + [vm((CHUNK, SIGNAL_DIM), jnp.float32)] * 3
            + [vm((CHUNK, SIGNAL_DIM), jnp.float32)]
        )
    sem_types = [pltpu.SemaphoreType.DMA] * (3 * nbuf)

    @functools.partial(
        pl.kernel,
        mesh=mesh,
        out_type=jax.ShapeDtypeStruct((p_n, SIGNAL_DIM), jnp.float32),
        scratch_types=buf_types + sem_types,
    )
    def k(table_hbm, i1_hbm, i2_hbm, i3_hbm, w1_hbm, w2_hbm, w3_hbm, out_hbm,
          *scr):
        wid = lax.axis_index("s") * 2 + lax.axis_index("c")
        i_hbm = (i1_hbm, i2_hbm, i3_hbm)
        w_hbm = (w1_hbm, w2_hbm, w3_hbm)
        bufs = [scr[b * 10:(b + 1) * 10] for b in range(nbuf)]
        sems = scr[10 * nbuf:]
        sem_idx = sems[0::3]
        sem_dat = sems[1::3]
        sem_out = sems[2::3]

        def load_idx(b, li):
            g = li * NW + wid

            @pl.when(g < nchunk)
            def _():
                c0 = g * CHUNK
                for t in range(3):
                    pltpu.async_copy(i_hbm[t].at[pl.ds(c0, CHUNK)],
                                     bufs[b][t], sem_idx[b])
                    pltpu.async_copy(w_hbm[t].at[pl.ds(c0, CHUNK)],
                                     bufs[b][3 + t], sem_dat[b])

        def launch_gather(b, li):
            g = li * NW + wid

            @pl.when(g < nchunk)
            def _():
                for t in range(3):
                    pltpu.make_async_copy(i_hbm[t].at[pl.ds(0, CHUNK)],
                                          bufs[b][t], sem_idx[b]).wait()
                for t in range(3):
                    pltpu.async_copy(table_hbm.at[bufs[b][t]],
                                     bufs[b][6 + t], sem_dat[b])

        def compute_store(b, li):
            g = li * NW + wid

            @pl.when(g < nchunk)
            def _():
                c0 = g * CHUNK
                x_v = bufs[b][9]
                r1, r2, r3 = bufs[b][6], bufs[b][7], bufs[b][8]
                for t in range(3):
                    pltpu.make_async_copy(w_hbm[t].at[pl.ds(0, CHUNK)],
                                          bufs[b][3 + t], sem_dat[b]).wait()
                for t in range(3):
                    pltpu.make_async_copy(table_hbm.at[bufs[b][t]],
                                          bufs[b][6 + t], sem_dat[b]).wait()

                @pl.when(li >= 2)
                def _():
                    pltpu.make_async_copy(
                        x_v, out_hbm.at[pl.ds(0, CHUNK)], sem_out[b]).wait()

                def group_body(gg, inner):
                    gbase = gg * 16
                    av1 = bufs[b][3][pl.ds(gbase, 16)]
                    av2 = bufs[b][4][pl.ds(gbase, 16)]
                    av3 = bufs[b][5][pl.ds(gbase, 16)]
                    for l in range(16):
                        c = gbase + l
                        a1 = av1[l]
                        a2 = av2[l]
                        a3 = av3[l]
                        for v in range(SIGNAL_DIM // 16):
                            sl = pl.ds(v * 16, 16)
                            x_v[c, sl] = (r1[c, sl] * a1 + r2[c, sl] * a2
                                          + r3[c, sl] * a3)
                    return inner

                lax.fori_loop(0, CHUNK // 16, group_body, None)
                pltpu.async_copy(x_v, out_hbm.at[pl.ds(c0, CHUNK)], sem_out[b])

        load_idx(0, 0)
        launch_gather(0, 0)
        load_idx(1, 1)
        launch_gather(1, 1)

        def outer_body(i, carry):
            li = i * 2
            compute_store(0, li)
            load_idx(0, li + 2)
            launch_gather(0, li + 2)
            compute_store(1, li + 1)
            load_idx(1, li + 3)
            launch_gather(1, li + 3)
            return carry

        lax.fori_loop(0, outer, outer_body, None)
        for b in range(nbuf):
            pltpu.make_async_copy(
                bufs[b][9], out_hbm.at[pl.ds(0, CHUNK)], sem_out[b]).wait()

    return k(table, i1, i2, i3, w1, w2, w3)


def _tc_conv(X, W):
    n = X.shape[0]
    M = 200
    grid = (n // M,)

    def body(x_ref, w_ref, o_ref):
        pre = jnp.dot(x_ref[...].astype(jnp.bfloat16),
                      w_ref[...].astype(jnp.bfloat16),
                      preferred_element_type=jnp.float32)
        pre = jnp.maximum(pre, 0.0)
        best = jnp.zeros((M,), jnp.int32)
        cur = None
        for r in range(N_ROT):
            pr = pre[:, r * 128:(r + 1) * 128]
            nr = jnp.sum(pr * pr, axis=-1)
            if cur is None:
                cur = nr
            else:
                gt = nr > cur
                best = jnp.where(gt, r, best)
                cur = jnp.maximum(nr, cur)
        out = jnp.zeros((M, 128), jnp.float32)
        for r in range(N_ROT):
            pr = pre[:, r * 128:(r + 1) * 128]
            out = out + jnp.where((best == r)[:, None], pr, 0.0)
        o_ref[...] = out

    return pl.pallas_call(
        body,
        grid=grid,
        in_specs=[
            pl.BlockSpec((M, KV * SIGNAL_DIM), lambda i: (i, 0)),
            pl.BlockSpec((KV * SIGNAL_DIM, N_ROT * 128), lambda i: (0, 0)),
        ],
        out_specs=pl.BlockSpec((M, 128), lambda i: (i, 0)),
        out_shape=jax.ShapeDtypeStruct((n, 128), jnp.float32),
    )(X, W)


def kernel(signal, b_coordinates, kernel):
    s = signal[0]
    b = b_coordinates[0].reshape(-1, 8)
    i1 = b[:, 3].astype(jnp.int32)
    i2 = b[:, 5].astype(jnp.int32)
    i3 = b[:, 7].astype(jnp.int32)
    w1 = b[:, 2]
    w2 = b[:, 4]
    w3 = b[:, 6]

    H = P // 2
    xs = [_sc_interp(s, i1[h * H:(h + 1) * H], i2[h * H:(h + 1) * H],
                     i3[h * H:(h + 1) * H], w1[h * H:(h + 1) * H],
                     w2[h * H:(h + 1) * H], w3[h * H:(h + 1) * H])
          for h in range(2)]
    Xs = [x.reshape(N_NODES // 2, KV * SIGNAL_DIM) for x in xs]

    krot = jnp.stack([jnp.roll(kernel, -r, axis=1) for r in range(N_ROT)], axis=0)
    krot = krot[:, :, :, 0]
    W = krot.transpose(1, 2, 4, 0, 3).reshape(KV * SIGNAL_DIM, N_ROT * 128)

    out = jnp.concatenate([_tc_conv(Xh, W) for Xh in Xs], axis=0)
    return out[None]


# speedup vs baseline: 9.2249x; 1.0032x over previous
import functools

import jax
import jax.numpy as jnp
from jax import lax
from jax.experimental import pallas as pl
from jax.experimental.pallas import tpu as pltpu
from jax.experimental.pallas import tpu_sc as plsc

N_NODES = 10000
SIGNAL_DIM = 128
KV = 16
N_ROT = 4
P = N_NODES * KV
NW = 32
CHUNK = 80
NCHUNK = P // CHUNK
ROUNDS = -(-NCHUNK // NW)
OUTER = (ROUNDS + 2) // 2


def _sc_interp(table, i1, i2, i3, w1, w2, w3):
    mesh = plsc.VectorSubcoreMesh(core_axis_name="c", subcore_axis_name="s")
    p_n = i1.shape[0]
    nchunk = p_n // CHUNK
    outer = (-(-nchunk // NW) + 1) // 2

    nbuf = 2
    vm = pltpu.VMEM
    buf_types = []
    for _ in range(nbuf):
        buf_types += (
            [vm((CHUNK,), jnp.int32)] * 3
            + [vm((CHUNK,), jnp.float32)] * 3
            + [vm((CHUNK, SIGNAL_DIM), jnp.float32)] * 3
            + [vm((CHUNK, SIGNAL_DIM), jnp.float32)]
        )
    sem_types = [pltpu.SemaphoreType.DMA] * (3 * nbuf)

    @functools.partial(
        pl.kernel,
        mesh=mesh,
        out_type=jax.ShapeDtypeStruct((p_n, SIGNAL_DIM), jnp.float32),
        scratch_types=buf_types + sem_types,
    )
    def k(table_hbm, i1_hbm, i2_hbm, i3_hbm, w1_hbm, w2_hbm, w3_hbm, out_hbm,
          *scr):
        wid = lax.axis_index("s") * 2 + lax.axis_index("c")
        i_hbm = (i1_hbm, i2_hbm, i3_hbm)
        w_hbm = (w1_hbm, w2_hbm, w3_hbm)
        bufs = [scr[b * 10:(b + 1) * 10] for b in range(nbuf)]
        sems = scr[10 * nbuf:]
        sem_idx = sems[0::3]
        sem_dat = sems[1::3]
        sem_out = sems[2::3]

        def load_idx(b, li):
            g = li * NW + wid

            @pl.when(g < nchunk)
            def _():
                c0 = g * CHUNK
                for t in range(3):
                    pltpu.async_copy(i_hbm[t].at[pl.ds(c0, CHUNK)],
                                     bufs[b][t], sem_idx[b])
                    pltpu.async_copy(w_hbm[t].at[pl.ds(c0, CHUNK)],
                                     bufs[b][3 + t], sem_dat[b])

        def launch_gather(b, li):
            g = li * NW + wid

            @pl.when(g < nchunk)
            def _():
                for t in range(3):
                    pltpu.make_async_copy(i_hbm[t].at[pl.ds(0, CHUNK)],
                                          bufs[b][t], sem_idx[b]).wait()
                for t in range(3):
                    pltpu.async_copy(table_hbm.at[bufs[b][t]],
                                     bufs[b][6 + t], sem_dat[b])

        def compute_store(b, li):
            g = li * NW + wid

            @pl.when(g < nchunk)
            def _():
                c0 = g * CHUNK
                x_v = bufs[b][9]
                r1, r2, r3 = bufs[b][6], bufs[b][7], bufs[b][8]
                for t in range(3):
                    pltpu.make_async_copy(w_hbm[t].at[pl.ds(0, CHUNK)],
                                          bufs[b][3 + t], sem_dat[b]).wait()
                for t in range(3):
                    pltpu.make_async_copy(table_hbm.at[bufs[b][t]],
                                          bufs[b][6 + t], sem_dat[b]).wait()

                @pl.when(li >= 2)
                def _():
                    pltpu.make_async_copy(
                        x_v, out_hbm.at[pl.ds(0, CHUNK)], sem_out[b]).wait()

                def group_body(gg, inner):
                    gbase = gg * 16
                    av1 = bufs[b][3][pl.ds(gbase, 16)]
                    av2 = bufs[b][4][pl.ds(gbase, 16)]
                    av3 = bufs[b][5][pl.ds(gbase, 16)]
                    for l in range(16):
                        c = gbase + l
                        a1 = av1[l]
                        a2 = av2[l]
                        a3 = av3[l]
                        for v in range(SIGNAL_DIM // 16):
                            sl = pl.ds(v * 16, 16)
                            x_v[c, sl] = (r1[c, sl] * a1 + r2[c, sl] * a2
                                          + r3[c, sl] * a3)
                    return inner

                lax.fori_loop(0, CHUNK // 16, group_body, None)
                pltpu.async_copy(x_v, out_hbm.at[pl.ds(c0, CHUNK)], sem_out[b])

        load_idx(0, 0)
        launch_gather(0, 0)
        load_idx(1, 1)
        launch_gather(1, 1)

        def outer_body(i, carry):
            li = i * 2
            compute_store(0, li)
            load_idx(0, li + 2)
            launch_gather(0, li + 2)
            compute_store(1, li + 1)
            load_idx(1, li + 3)
            launch_gather(1, li + 3)
            return carry

        lax.fori_loop(0, outer, outer_body, None)
        for b in range(nbuf):
            pltpu.make_async_copy(
                bufs[b][9], out_hbm.at[pl.ds(0, CHUNK)], sem_out[b]).wait()

    return k(table, i1, i2, i3, w1, w2, w3)


def _tc_conv(X, W):
    n = X.shape[0]
    M = 200
    grid = (n // M,)

    def body(x_ref, w_ref, o_ref):
        pre = jnp.dot(x_ref[...].astype(jnp.bfloat16),
                      w_ref[...].astype(jnp.bfloat16),
                      preferred_element_type=jnp.float32)
        pre = jnp.maximum(pre, 0.0)
        best = jnp.zeros((M,), jnp.int32)
        cur = None
        for r in range(N_ROT):
            pr = pre[:, r * 128:(r + 1) * 128]
            nr = jnp.sum(pr * pr, axis=-1)
            if cur is None:
                cur = nr
            else:
                gt = nr > cur
                best = jnp.where(gt, r, best)
                cur = jnp.maximum(nr, cur)
        out = jnp.zeros((M, 128), jnp.float32)
        for r in range(N_ROT):
            pr = pre[:, r * 128:(r + 1) * 128]
            out = out + jnp.where((best == r)[:, None], pr, 0.0)
        o_ref[...] = out

    return pl.pallas_call(
        body,
        grid=grid,
        in_specs=[
            pl.BlockSpec((M, KV * SIGNAL_DIM), lambda i: (i, 0)),
            pl.BlockSpec((KV * SIGNAL_DIM, N_ROT * 128), lambda i: (0, 0)),
        ],
        out_specs=pl.BlockSpec((M, 128), lambda i: (i, 0)),
        out_shape=jax.ShapeDtypeStruct((n, 128), jnp.float32),
    )(X, W)


def kernel(signal, b_coordinates, kernel):
    s = signal[0]
    b = b_coordinates[0].reshape(-1, 8)
    i1 = b[:, 3].astype(jnp.int32)
    i2 = b[:, 5].astype(jnp.int32)
    i3 = b[:, 7].astype(jnp.int32)
    w1 = b[:, 2]
    w2 = b[:, 4]
    w3 = b[:, 6]

    H = P // 4
    xs = [_sc_interp(s, i1[h * H:(h + 1) * H], i2[h * H:(h + 1) * H],
                     i3[h * H:(h + 1) * H], w1[h * H:(h + 1) * H],
                     w2[h * H:(h + 1) * H], w3[h * H:(h + 1) * H])
          for h in range(4)]
    Xs = [x.reshape(N_NODES // 4, KV * SIGNAL_DIM) for x in xs]

    krot = jnp.stack([jnp.roll(kernel, -r, axis=1) for r in range(N_ROT)], axis=0)
    krot = krot[:, :, :, 0]
    W = krot.transpose(1, 2, 4, 0, 3).reshape(KV * SIGNAL_DIM, N_ROT * 128)

    out = jnp.concatenate([_tc_conv(Xh, W) for Xh in Xs], axis=0)
    return out[None]
